# Initial kernel scaffold; baseline (speedup 1.0000x reference)
import jax
import jax.numpy as jnp
from jax.experimental import pallas as pl


def kernel(inputs, codebook):
    raise NotImplementedError("write your pallas kernel here")


# speedup vs baseline: 7.8174x; 7.8174x over previous
import jax
import jax.numpy as jnp
from jax import lax
from jax.experimental import pallas as pl

_N = 512
_K = 1024
_D = 256
_NCAND = 4


def _refined_dist(xt, rowt):
    sq = (xt - rowt) * (xt - rowt)
    totals = []
    for c in range(2):
        r = sq[128 * c:128 * (c + 1), :].reshape(16, 8, _N)
        p = r[0]
        for k in range(1, 16):
            p = p + r[k]
        a = ((p[0:1] + p[4:5]) + (p[2:3] + p[6:7])) + (
            (p[1:2] + p[5:6]) + (p[3:4] + p[7:8]))
        totals.append(a)
    return (totals[0] + totals[1]) * jnp.float32(1.0 / _D)


def _vq_kernel(xt_ref, cb_ref, loss_ref, qt_ref, idx_ref):
    xt = xt_ref[...]
    cb = cb_ref[...]

    xct = lax.dot_general(
        cb, xt,
        dimension_numbers=(((1,), (0,)), ((), ())),
        preferred_element_type=jnp.float32,
        precision=lax.Precision.HIGHEST,
    )
    cb2 = cb * cb
    ones = jnp.ones((_D, 1), jnp.float32)
    cnorm = lax.dot_general(
        cb2, ones,
        dimension_numbers=(((1,), (0,)), ((), ())),
        preferred_element_type=jnp.float32,
        precision=lax.Precision.HIGHEST,
    )
    st = cnorm - 2.0 * xct

    riota = lax.broadcasted_iota(jnp.int32, (_K, _N), 0)

    cand_idx = []
    for _ in range(_NCAND):
        m = jnp.min(st, axis=0, keepdims=True)
        i = jnp.min(jnp.where(st == m, riota, _K), axis=0,
                    keepdims=True).astype(jnp.int32)
        cand_idx.append(i)
        st = jnp.where(riota == i, jnp.inf, st)

    best_d = None
    best_i = None
    best_qt = None
    for c in range(_NCAND):
        i_c = cand_idx[c]
        oht = (riota == i_c).astype(jnp.float32)
        rowt = lax.dot_general(
            cb, oht,
            dimension_numbers=(((0,), (0,)), ((), ())),
            preferred_element_type=jnp.float32,
            precision=lax.Precision.HIGHEST,
        )
        d = _refined_dist(xt, rowt)
        if best_d is None:
            best_d, best_i, best_qt = d, i_c, rowt
        else:
            lt = (d < best_d) | ((d == best_d) & (i_c < best_i))
            best_d = jnp.where(lt, d, best_d)
            best_i = jnp.where(lt, i_c, best_i)
            best_qt = jnp.where(lt, rowt, best_qt)

    diff = best_qt - xt
    m2 = jnp.sum(diff * diff) * jnp.float32(1.0 / (_N * _D))
    loss_ref[...] = (m2 + jnp.float32(0.25) * m2)[None, None]
    qt_ref[...] = xt + (best_qt - xt)
    idx_ref[...] = best_i


@jax.jit
def kernel(inputs, codebook):
    xt = jnp.transpose(inputs, (0, 2, 3, 1)).reshape(_N, _D).T
    loss, qt, idx = pl.pallas_call(
        _vq_kernel,
        out_shape=(
            jax.ShapeDtypeStruct((1, 1), jnp.float32),
            jax.ShapeDtypeStruct((_D, _N), jnp.float32),
            jax.ShapeDtypeStruct((1, _N), jnp.int32),
        ),
    )(xt, codebook)
    quantized = jnp.transpose(qt.reshape(_D, 2, 16, 16), (1, 0, 2, 3))
    return loss.reshape(()), quantized, idx.reshape(2, 256)
